# baseline (speedup 1.0000x reference)
import functools

import jax
import jax.numpy as jnp
from jax import lax
from jax.experimental import pallas as pl
from jax.experimental.pallas import tpu as pltpu
from jax.experimental.pallas import tpu_sc as plsc

N_ENT = 1000000
N_REL = 1000
D = 64
B = 16384

NC = 2
NS = 16
NW = NC * NS
BPW = B // NW
CHUNK = 128
NCHUNK = BPW // CHUNK
GROUPS = BPW // 16


@functools.partial(
    pl.kernel,
    out_type=jax.ShapeDtypeStruct((B,), jnp.float32),
    mesh=plsc.VectorSubcoreMesh(core_axis_name="c", subcore_axis_name="s"),
    compiler_params=pltpu.CompilerParams(
        use_tc_tiling_on_sc=False, needs_layout_passes=False),
    scratch_types=[
        pltpu.VMEM((NCHUNK, CHUNK), jnp.int32),
        pltpu.VMEM((NCHUNK, CHUNK), jnp.int32),
        pltpu.VMEM((NCHUNK, CHUNK), jnp.int32),
        pltpu.VMEM((BPW, D), jnp.float32),
        pltpu.VMEM((BPW, D), jnp.float32),
        pltpu.VMEM((BPW, D), jnp.float32),
        pltpu.VMEM((BPW,), jnp.float32),
        pltpu.SemaphoreType.DMA,
    ],
)
def _kge_sc(src_hbm, tgt_hbm, rel_hbm, ent_hbm, relt_hbm, out_hbm,
            idx_s, idx_t, idx_r, s_rows, t_rows, r_rows, out_v, sem):
    wid = lax.axis_index("s") * NC + lax.axis_index("c")
    base = wid * BPW

    for c in range(NCHUNK):
        off = base + c * CHUNK
        pltpu.sync_copy(src_hbm.at[pl.ds(off, CHUNK)], idx_s.at[c])
        pltpu.sync_copy(tgt_hbm.at[pl.ds(off, CHUNK)], idx_t.at[c])
        pltpu.sync_copy(rel_hbm.at[pl.ds(off, CHUNK)], idx_r.at[c])

    copies = []
    for c in range(NCHUNK):
        dst = pl.ds(c * CHUNK, CHUNK)
        copies.append(pltpu.async_copy(ent_hbm.at[idx_s.at[c]], s_rows.at[dst], sem))
        copies.append(pltpu.async_copy(ent_hbm.at[idx_t.at[c]], t_rows.at[dst], sem))
        copies.append(pltpu.async_copy(relt_hbm.at[idx_r.at[c]], r_rows.at[dst], sem))
    for cp in copies:
        cp.wait()

    lane = lax.iota(jnp.int32, 16)

    def group_body(g, _):
        row = g * 16 + lane

        def dim_body(j, acc):
            col = jnp.full((16,), j, dtype=jnp.int32)
            sv = plsc.load_gather(s_rows, [row, col])
            tv = plsc.load_gather(t_rows, [row, col])
            rv = plsc.load_gather(r_rows, [row, col])
            return acc + sv * tv * rv

        acc = lax.fori_loop(0, D, dim_body, jnp.zeros((16,), jnp.float32))
        out_v[pl.ds(g * 16, 16)] = acc
        return 0

    lax.fori_loop(0, GROUPS, group_body, 0)

    pltpu.sync_copy(out_v, out_hbm.at[pl.ds(base, BPW)])


def kernel(sources, targets, rels, ent_table, rel_table):
    return _kge_sc(sources.astype(jnp.int32), targets.astype(jnp.int32),
                   rels.astype(jnp.int32), ent_table, rel_table)


# speedup vs baseline: 1.6577x; 1.6577x over previous
import functools

import jax
import jax.numpy as jnp
from jax import lax
from jax.experimental import pallas as pl
from jax.experimental.pallas import tpu as pltpu
from jax.experimental.pallas import tpu_sc as plsc

N_ENT = 1000000
N_REL = 1000
D = 64
B = 16384

NC = 2
NS = 16
NW = NC * NS
BPW = B // NW
GROUPS = BPW // 16
ROW_BYTES = D * 4


@functools.partial(
    pl.kernel,
    out_type=jax.ShapeDtypeStruct((B,), jnp.float32),
    mesh=plsc.VectorSubcoreMesh(core_axis_name="c", subcore_axis_name="s"),
    compiler_params=pltpu.CompilerParams(needs_layout_passes=False),
    scratch_types=[
        pltpu.VMEM((BPW,), jnp.int32),
        pltpu.VMEM((BPW,), jnp.int32),
        pltpu.VMEM((BPW,), jnp.int32),
        pltpu.VMEM((BPW // 2, 2 * D), jnp.float32),
        pltpu.VMEM((BPW // 2, 2 * D), jnp.float32),
        pltpu.VMEM((BPW // 2, 2 * D), jnp.float32),
        pltpu.VMEM((BPW,), jnp.float32),
        pltpu.SemaphoreType.DMA,
    ],
)
def _kge_sc(src_hbm, tgt_hbm, rel_hbm, ent_hbm, relt_hbm, drain_hbm, out_hbm,
            idx_s, idx_t, idx_r, s_rows, t_rows, r_rows, out_v, sem):
    wid = lax.axis_index("s") * NC + lax.axis_index("c")
    base = wid * BPW

    pltpu.sync_copy(src_hbm.at[pl.ds(base, BPW)], idx_s)
    pltpu.sync_copy(tgt_hbm.at[pl.ds(base, BPW)], idx_t)
    pltpu.sync_copy(rel_hbm.at[pl.ds(base, BPW)], idx_r)

    def issue_body(g, _):
        vs = idx_s[pl.ds(g * 16, 16)]
        vt = idx_t[pl.ds(g * 16, 16)]
        vr = idx_r[pl.ds(g * 16, 16)]
        for k in range(16):
            r2 = g * 8 + (k // 2)
            cds = pl.ds((k % 2) * D, D)
            pltpu.async_copy(ent_hbm.at[vs[k]], s_rows.at[r2, cds], sem)
            pltpu.async_copy(ent_hbm.at[vt[k]], t_rows.at[r2, cds], sem)
            pltpu.async_copy(relt_hbm.at[vr[k]], r_rows.at[r2, cds], sem)
        return 0

    lax.fori_loop(0, GROUPS, issue_body, 0)

    pltpu.make_async_copy(drain_hbm, s_rows, sem).wait()
    pltpu.make_async_copy(drain_hbm, t_rows, sem).wait()
    pltpu.make_async_copy(drain_hbm, r_rows, sem).wait()

    lane = lax.iota(jnp.int32, 16)
    lane_half = lax.shift_right_logical(lane, 1)
    colbase = lax.mul(lax.bitwise_and(lane, jnp.ones((16,), jnp.int32)),
                      jnp.full((16,), D, jnp.int32))

    def group_body(g, _):
        rowhalf = g * 8 + lane_half

        def dim_body(j, acc):
            col = colbase + j
            sv = plsc.load_gather(s_rows, [rowhalf, col])
            tv = plsc.load_gather(t_rows, [rowhalf, col])
            rv = plsc.load_gather(r_rows, [rowhalf, col])
            return acc + sv * tv * rv

        acc = lax.fori_loop(0, D, dim_body, jnp.zeros((16,), jnp.float32),
                            unroll=8)
        out_v[pl.ds(g * 16, 16)] = acc
        return 0

    lax.fori_loop(0, GROUPS, group_body, 0)

    pltpu.sync_copy(out_v, out_hbm.at[pl.ds(base, BPW)])


def kernel(sources, targets, rels, ent_table, rel_table):
    drain = jnp.zeros((BPW // 2, 2 * D), jnp.float32)
    return _kge_sc(sources.astype(jnp.int32), targets.astype(jnp.int32),
                   rels.astype(jnp.int32), ent_table, rel_table, drain)
